# Initial kernel scaffold; baseline (speedup 1.0000x reference)
import jax
import jax.numpy as jnp
from jax.experimental import pallas as pl


def kernel(src, tgt, part_indicator):
    raise NotImplementedError("write your pallas kernel here")


# speedup vs baseline: 1.0712x; 1.0712x over previous
import functools

import jax
import jax.numpy as jnp
from jax import lax
from jax.experimental import pallas as pl
from jax.experimental.pallas import tpu as pltpu
from jax.experimental.pallas import tpu_sc as plsc

B, P, D = 4096, 128, 512
NC, NS, L = 2, 16, 16
NW = NC * NS
BPW = B // NW
GROUPS = BPW // L
PV = P // L


def _make_kernel():
    mesh = plsc.VectorSubcoreMesh(core_axis_name="c", subcore_axis_name="s")

    @functools.partial(
        pl.kernel,
        mesh=mesh,
        compiler_params=pltpu.CompilerParams(needs_layout_passes=False),
        out_type=jax.ShapeDtypeStruct((B, D), jnp.float32),
        scratch_types=[
            pltpu.VMEM((BPW, P), jnp.float32),
            pltpu.VMEM((BPW,), jnp.int32),
            pltpu.VMEM((BPW, D), jnp.float32),
            pltpu.SemaphoreType.DMA,
        ],
    )
    def sel(tgt_hbm, pi_hbm, out_hbm, pi_v, idx_v, rows_v, sem):
        wid = lax.axis_index("s") * NC + lax.axis_index("c")
        base = wid * BPW

        pltpu.sync_copy(pi_hbm.at[pl.ds(base, BPW)], pi_v)

        lane = lax.iota(jnp.int32, L)

        def group_body(g, carry):
            res = jnp.zeros((L,), jnp.int32)
            for i in range(L):
                b = g * L + i
                m = pi_v[b, pl.ds(0, L)]
                a = jnp.zeros((L,), jnp.int32)
                for j in range(1, PV):
                    v = pi_v[b, pl.ds(j * L, L)]
                    a = jnp.where(v > m, j, a)
                    m = jnp.maximum(m, v)
                mx = jnp.max(m)
                cand = jnp.where(m == mx, a * L + lane, P)
                p_sel = jnp.min(cand)
                row = (base + b) * P + p_sel
                res = jnp.where(lane == i, row, res)
            idx_v[pl.ds(g * L, L)] = res
            return carry

        lax.fori_loop(0, GROUPS, group_body, 0)

        pltpu.async_copy(tgt_hbm.at[idx_v], rows_v, sem).wait()
        pltpu.sync_copy(rows_v, out_hbm.at[pl.ds(base, BPW)])

    return sel


_SEL = _make_kernel()


@jax.jit
def kernel(src, tgt, part_indicator):
    del src
    out = _SEL(tgt.reshape(B * P, D), part_indicator)
    return out.reshape(B, 1, D)


# speedup vs baseline: 1.4469x; 1.3507x over previous
import functools

import jax
import jax.numpy as jnp
from jax import lax
from jax.experimental import pallas as pl
from jax.experimental.pallas import tpu as pltpu
from jax.experimental.pallas import tpu_sc as plsc

B, P, D = 4096, 128, 512
NC, NS, L = 2, 16, 16
NW = NC * NS
BPW = B // NW
GROUPS = BPW // L
PV = P // L


def _make_kernel():
    mesh = plsc.VectorSubcoreMesh(core_axis_name="c", subcore_axis_name="s")

    @functools.partial(
        pl.kernel,
        mesh=mesh,
        compiler_params=pltpu.CompilerParams(needs_layout_passes=False),
        out_type=jax.ShapeDtypeStruct((B, 1, D), jnp.float32),
        scratch_types=[
            pltpu.VMEM((BPW, P), jnp.float32),
            pltpu.VMEM((BPW,), jnp.int32),
            pltpu.VMEM((BPW, 1, D), jnp.float32),
            pltpu.SemaphoreType.DMA,
        ],
    )
    def sel(tgt_hbm, pi_hbm, out_hbm, pi_v, idx_v, rows_v, sem):
        wid = lax.axis_index("s") * NC + lax.axis_index("c")
        base = wid * BPW

        pltpu.sync_copy(pi_hbm.at[pl.ds(base, BPW)], pi_v)

        lane = lax.iota(jnp.int32, L)

        def group_body(g, carry):
            res = jnp.zeros((L,), jnp.int32)
            for i in range(L):
                b = g * L + i
                m = pi_v[b, pl.ds(0, L)]
                a = jnp.zeros((L,), jnp.int32)
                for j in range(1, PV):
                    v = pi_v[b, pl.ds(j * L, L)]
                    a = jnp.where(v > m, j, a)
                    m = jnp.maximum(m, v)
                mx = jnp.max(m)
                cand = jnp.where(m == mx, a * L + lane, P)
                p_sel = jnp.min(cand)
                row = (base + b) * P + p_sel
                res = jnp.where(lane == i, row, res)
            idx_v[pl.ds(g * L, L)] = res
            return carry

        lax.fori_loop(0, GROUPS, group_body, 0)

        pltpu.async_copy(tgt_hbm.at[idx_v], rows_v.at[:, 0], sem).wait()
        pltpu.sync_copy(rows_v, out_hbm.at[pl.ds(base, BPW)])

    return sel


_SEL = _make_kernel()


@jax.jit
def kernel(src, tgt, part_indicator):
    del src
    return _SEL(tgt.reshape(B * P, D), part_indicator)
